# baseline (speedup 1.0000x reference)
import jax
import jax.numpy as jnp
from jax import lax
from jax.experimental import pallas as pl
from jax.experimental.pallas import tpu as pltpu
from jax.experimental.pallas import tpu_sc as plsc

_K = 1024
_D = 64
_N = 16384
_COMMIT = 0.25

_BT = 512
_NB = _N // _BT

_NC, _NS = 2, 16
_NW = _NC * _NS
_BW = _N // _NW
_CH = 128
_NCH = _BW // _CH


def _vq_tc_body(x_ref, c_ref, idx_ref, acc_ref, cn_ref):
    x = x_ref[...]
    c = c_ref[...]
    rn = jnp.sum(x * x, axis=1, keepdims=True)

    @pl.when(pl.program_id(0) == 0)
    def _cn_once():
        cn_ref[...] = jnp.sum(c * c, axis=1)[None, :]

    cn = cn_ref[0, :]
    mm = lax.dot_general(x, c, (((1,), (1,)), ((), ())),
                         preferred_element_type=jnp.float32)
    d = (rn + cn[None, :]) - 2.0 * mm
    lane = lax.broadcasted_iota(jnp.int32, (128, 128), 1).astype(jnp.float32)
    idx_parts = []
    gsum = None
    for g in range(_BT // 128):
        r0 = g * 128
        bestv = lax.slice(d, (r0, 0), (r0 + 128, 128))
        bestc = jnp.zeros((128, 128), jnp.float32)
        for ci in range(1, _K // 128):
            dc = lax.slice(d, (r0, ci * 128), (r0 + 128, (ci + 1) * 128))
            m = dc < bestv
            bestv = jnp.minimum(dc, bestv)
            bestc = jnp.where(m, float(ci), bestc)
        cand = bestc * 128.0 + lane
        bvt = bestv.T
        gmin_t = jnp.min(bvt, axis=0, keepdims=True)
        idxg = jnp.min(jnp.where(bvt == gmin_t, cand.T, float(_K)), axis=0)
        idx_parts.append(idxg)
        s = jnp.sum(gmin_t)
        gsum = s if gsum is None else gsum + s
    idx_f = jnp.concatenate(idx_parts)
    idx_ref[...] = idx_f.astype(jnp.int32)

    @pl.when(pl.program_id(0) == 0)
    def _init():
        acc_ref[...] = jnp.zeros_like(acc_ref)

    acc_ref[...] += gsum.reshape(1, 1)


_vq_tc = pl.pallas_call(
    _vq_tc_body,
    grid=(_NB,),
    in_specs=[
        pl.BlockSpec((_BT, _D), lambda i: (i, 0)),
        pl.BlockSpec((_K, _D), lambda i: (0, 0)),
    ],
    out_specs=[
        pl.BlockSpec((_BT,), lambda i: (i,)),
        pl.BlockSpec((1, 1), lambda i: (0, 0)),
    ],
    out_shape=[
        jax.ShapeDtypeStruct((_N,), jnp.int32),
        jax.ShapeDtypeStruct((1, 1), jnp.float32),
    ],
    scratch_shapes=[pltpu.VMEM((1, _K), jnp.float32)],
    compiler_params=pltpu.CompilerParams(dimension_semantics=("arbitrary",)),
)


def _sc_gather_body(table_hbm, idx_hbm, out_hbm, idx_v, rows_v, sem):
    wid = lax.axis_index("s") * _NC + lax.axis_index("c")
    base = wid * _BW
    pltpu.sync_copy(idx_hbm.at[pl.ds(base, _BW)], idx_v)
    copies = [
        pltpu.async_copy(table_hbm.at[idx_v.at[pl.ds(j * _CH, _CH)]],
                         rows_v.at[j], sem)
        for j in range(_NCH)
    ]
    for j, cp in enumerate(copies):
        cp.wait()
        pltpu.sync_copy(rows_v.at[j],
                        out_hbm.at[pl.ds(base + j * _CH, _CH)])


import functools


@functools.lru_cache(maxsize=1)
def _sc_gather():
    return pl.kernel(
        _sc_gather_body,
        mesh=plsc.VectorSubcoreMesh(core_axis_name="c", subcore_axis_name="s"),
        out_type=jax.ShapeDtypeStruct((_N, 2 * _D), jnp.float32),
        scratch_types=[
            pltpu.VMEM((_BW,), jnp.int32),
            pltpu.VMEM((_NCH, _CH, 2 * _D), jnp.float32),
            pltpu.SemaphoreType.DMA,
        ],
    )


def kernel(inputs, codebook):
    idx, acc = _vq_tc(inputs, codebook)
    cb_pad = jnp.pad(codebook, ((0, 0), (0, _D)))
    quantized = _sc_gather()(cb_pad, idx)[:, :_D]
    mean_d = acc[0, 0] / (_N * _D)
    loss = mean_d + _COMMIT * mean_d
    return quantized, loss, idx


# speedup vs baseline: 1.0282x; 1.0282x over previous
import jax
import jax.numpy as jnp
from jax import lax
from jax.experimental import pallas as pl
from jax.experimental.pallas import tpu as pltpu
from jax.experimental.pallas import tpu_sc as plsc

_K = 1024
_D = 64
_N = 16384
_COMMIT = 0.25

_BT = 512
_NB = _N // _BT

_NC, _NS = 2, 16
_NW = _NC * _NS
_BW = _N // _NW
_CH = 128
_NCH = _BW // _CH


def _vq_tc_body(x_ref, c_ref, cn_ref, idx_ref, acc_ref):
    x = x_ref[...]
    c = c_ref[...]
    rn = jnp.sum(x * x, axis=1, keepdims=True)
    cn = cn_ref[0, :]
    mm = lax.dot_general(x, c, (((1,), (1,)), ((), ())),
                         preferred_element_type=jnp.float32)
    d = (rn + cn[None, :]) - 2.0 * mm
    lane = lax.broadcasted_iota(jnp.int32, (128, 128), 1).astype(jnp.float32)
    idx_parts = []
    gsum = None
    for g in range(_BT // 128):
        r0 = g * 128
        bestv = lax.slice(d, (r0, 0), (r0 + 128, 128))
        bestc = jnp.zeros((128, 128), jnp.float32)
        for ci in range(1, _K // 128):
            dc = lax.slice(d, (r0, ci * 128), (r0 + 128, (ci + 1) * 128))
            m = dc < bestv
            bestv = jnp.minimum(dc, bestv)
            bestc = jnp.where(m, float(ci), bestc)
        cand = bestc * 128.0 + lane
        bvt = bestv.T
        gmin_t = jnp.min(bvt, axis=0, keepdims=True)
        idxg = jnp.min(jnp.where(bvt == gmin_t, cand.T, float(_K)), axis=0)
        idx_parts.append(idxg)
        s = jnp.sum(gmin_t)
        gsum = s if gsum is None else gsum + s
    idx_f = jnp.concatenate(idx_parts)
    idx_ref[...] = idx_f.astype(jnp.int32)

    @pl.when(pl.program_id(0) == 0)
    def _init():
        acc_ref[...] = jnp.zeros_like(acc_ref)

    acc_ref[...] += gsum.reshape(1, 1)


_vq_tc = pl.pallas_call(
    _vq_tc_body,
    grid=(_NB,),
    in_specs=[
        pl.BlockSpec((_BT, _D), lambda i: (i, 0)),
        pl.BlockSpec((_K, _D), lambda i: (0, 0)),
        pl.BlockSpec((1, _K), lambda i: (0, 0)),
    ],
    out_specs=[
        pl.BlockSpec((_BT,), lambda i: (i,)),
        pl.BlockSpec((1, 1), lambda i: (0, 0)),
    ],
    out_shape=[
        jax.ShapeDtypeStruct((_N,), jnp.int32),
        jax.ShapeDtypeStruct((1, 1), jnp.float32),
    ],
    compiler_params=pltpu.CompilerParams(dimension_semantics=("arbitrary",)),
)


def _sc_gather_body(table_hbm, idx_hbm, out_hbm, idx_v, rows_v, sem):
    wid = lax.axis_index("s") * _NC + lax.axis_index("c")
    base = wid * _BW
    pltpu.sync_copy(idx_hbm.at[pl.ds(base, _BW)], idx_v)
    copies = [
        pltpu.async_copy(table_hbm.at[idx_v.at[pl.ds(j * _CH, _CH)]],
                         rows_v.at[j], sem)
        for j in range(_NCH)
    ]
    for j, cp in enumerate(copies):
        cp.wait()
        pltpu.sync_copy(rows_v.at[j],
                        out_hbm.at[pl.ds(base + j * _CH, _CH)])


import functools


@functools.lru_cache(maxsize=1)
def _sc_gather():
    return pl.kernel(
        _sc_gather_body,
        mesh=plsc.VectorSubcoreMesh(core_axis_name="c", subcore_axis_name="s"),
        out_type=jax.ShapeDtypeStruct((_N, 2 * _D), jnp.float32),
        scratch_types=[
            pltpu.VMEM((_BW,), jnp.int32),
            pltpu.VMEM((_NCH, _CH, 2 * _D), jnp.float32),
            pltpu.SemaphoreType.DMA,
        ],
    )


def kernel(inputs, codebook):
    cn = jnp.sum(codebook ** 2, axis=1)[None, :]
    idx, acc = _vq_tc(inputs, codebook, cn)
    cb_pad = jnp.pad(codebook, ((0, 0), (0, _D)))
    quantized = _sc_gather()(cb_pad, idx)[:, :_D]
    mean_d = acc[0, 0] / (_N * _D)
    loss = mean_d + _COMMIT * mean_d
    return quantized, loss, idx


# speedup vs baseline: 1.1096x; 1.0791x over previous
import jax
import jax.numpy as jnp
from jax import lax
from jax.experimental import pallas as pl
from jax.experimental.pallas import tpu as pltpu
from jax.experimental.pallas import tpu_sc as plsc

_K = 1024
_D = 64
_N = 16384
_COMMIT = 0.25

_BT = 1024
_NB = _N // _BT

_NC, _NS = 2, 16
_NW = _NC * _NS
_BW = _N // _NW
_CH = 128
_NCH = _BW // _CH


def _vq_tc_body(x_ref, c_ref, cn_ref, idx_ref, acc_ref):
    x = x_ref[...]
    c = c_ref[...]
    rn = jnp.sum(x * x, axis=1, keepdims=True)
    cn = cn_ref[0, :]
    mm = lax.dot_general(x, c, (((1,), (1,)), ((), ())),
                         preferred_element_type=jnp.float32)
    d = (rn + cn[None, :]) - 2.0 * mm
    lane = lax.broadcasted_iota(jnp.int32, (128, 128), 1).astype(jnp.float32)
    idx_parts = []
    gsum = None
    for g in range(_BT // 128):
        r0 = g * 128
        bestv = lax.slice(d, (r0, 0), (r0 + 128, 128))
        bestc = jnp.zeros((128, 128), jnp.float32)
        for ci in range(1, _K // 128):
            dc = lax.slice(d, (r0, ci * 128), (r0 + 128, (ci + 1) * 128))
            m = dc < bestv
            bestv = jnp.minimum(dc, bestv)
            bestc = jnp.where(m, float(ci), bestc)
        cand = bestc * 128.0 + lane
        bvt = bestv.T
        gmin_t = jnp.min(bvt, axis=0, keepdims=True)
        idxg = jnp.min(jnp.where(bvt == gmin_t, cand.T, float(_K)), axis=0)
        idx_parts.append(idxg)
        s = jnp.sum(gmin_t)
        gsum = s if gsum is None else gsum + s
    idx_f = jnp.concatenate(idx_parts)
    idx_ref[...] = idx_f.astype(jnp.int32)

    @pl.when(pl.program_id(0) == 0)
    def _init():
        acc_ref[...] = jnp.zeros_like(acc_ref)

    acc_ref[...] += gsum.reshape(1, 1)


_vq_tc = pl.pallas_call(
    _vq_tc_body,
    grid=(_NB,),
    in_specs=[
        pl.BlockSpec((_BT, _D), lambda i: (i, 0)),
        pl.BlockSpec((_K, _D), lambda i: (0, 0)),
        pl.BlockSpec((1, _K), lambda i: (0, 0)),
    ],
    out_specs=[
        pl.BlockSpec((_BT,), lambda i: (i,)),
        pl.BlockSpec((1, 1), lambda i: (0, 0)),
    ],
    out_shape=[
        jax.ShapeDtypeStruct((_N,), jnp.int32),
        jax.ShapeDtypeStruct((1, 1), jnp.float32),
    ],
    compiler_params=pltpu.CompilerParams(dimension_semantics=("arbitrary",)),
)


def _sc_gather_body(table_hbm, idx_hbm, out_hbm, idx_v, rows_v, sem):
    wid = lax.axis_index("s") * _NC + lax.axis_index("c")
    base = wid * _BW
    pltpu.sync_copy(idx_hbm.at[pl.ds(base, _BW)], idx_v)
    copies = [
        pltpu.async_copy(table_hbm.at[idx_v.at[pl.ds(j * _CH, _CH)]],
                         rows_v.at[j], sem)
        for j in range(_NCH)
    ]
    for j, cp in enumerate(copies):
        cp.wait()
        pltpu.sync_copy(rows_v.at[j],
                        out_hbm.at[pl.ds(base + j * _CH, _CH)])


import functools


@functools.lru_cache(maxsize=1)
def _sc_gather():
    return pl.kernel(
        _sc_gather_body,
        mesh=plsc.VectorSubcoreMesh(core_axis_name="c", subcore_axis_name="s"),
        out_type=jax.ShapeDtypeStruct((_N, 2 * _D), jnp.float32),
        scratch_types=[
            pltpu.VMEM((_BW,), jnp.int32),
            pltpu.VMEM((_NCH, _CH, 2 * _D), jnp.float32),
            pltpu.SemaphoreType.DMA,
        ],
    )


def kernel(inputs, codebook):
    cn = jnp.sum(codebook ** 2, axis=1)[None, :]
    idx, acc = _vq_tc(inputs, codebook, cn)
    cb_pad = jnp.pad(codebook, ((0, 0), (0, _D)))
    quantized = _sc_gather()(cb_pad, idx)[:, :_D]
    mean_d = acc[0, 0] / (_N * _D)
    loss = mean_d + _COMMIT * mean_d
    return quantized, loss, idx


# speedup vs baseline: 1.1505x; 1.0369x over previous
import jax
import jax.numpy as jnp
from jax import lax
from jax.experimental import pallas as pl
from jax.experimental.pallas import tpu as pltpu
from jax.experimental.pallas import tpu_sc as plsc

_K = 1024
_D = 64
_N = 16384
_COMMIT = 0.25

_BT = 2048
_NB = _N // _BT

_NC, _NS = 2, 16
_NW = _NC * _NS
_BW = _N // _NW
_CH = 128
_NCH = _BW // _CH


def _vq_tc_body(x_ref, c_ref, cn_ref, idx_ref, acc_ref):
    x = x_ref[...]
    c = c_ref[...]
    rn = jnp.sum(x * x, axis=1, keepdims=True)
    cn = cn_ref[0, :]
    mm = lax.dot_general(x, c, (((1,), (1,)), ((), ())),
                         preferred_element_type=jnp.float32)
    d = (rn + cn[None, :]) - 2.0 * mm
    lane = lax.broadcasted_iota(jnp.int32, (128, 128), 1).astype(jnp.float32)
    idx_parts = []
    gsum = None
    for g in range(_BT // 128):
        r0 = g * 128
        bestv = lax.slice(d, (r0, 0), (r0 + 128, 128))
        bestc = jnp.zeros((128, 128), jnp.float32)
        for ci in range(1, _K // 128):
            dc = lax.slice(d, (r0, ci * 128), (r0 + 128, (ci + 1) * 128))
            m = dc < bestv
            bestv = jnp.minimum(dc, bestv)
            bestc = jnp.where(m, float(ci), bestc)
        cand = bestc * 128.0 + lane
        bvt = bestv.T
        gmin_t = jnp.min(bvt, axis=0, keepdims=True)
        idxg = jnp.min(jnp.where(bvt == gmin_t, cand.T, float(_K)), axis=0)
        idx_parts.append(idxg)
        s = jnp.sum(gmin_t)
        gsum = s if gsum is None else gsum + s
    idx_f = jnp.concatenate(idx_parts)
    idx_ref[...] = idx_f.astype(jnp.int32)

    @pl.when(pl.program_id(0) == 0)
    def _init():
        acc_ref[...] = jnp.zeros_like(acc_ref)

    acc_ref[...] += gsum.reshape(1, 1)


_vq_tc = pl.pallas_call(
    _vq_tc_body,
    grid=(_NB,),
    in_specs=[
        pl.BlockSpec((_BT, _D), lambda i: (i, 0)),
        pl.BlockSpec((_K, _D), lambda i: (0, 0)),
        pl.BlockSpec((1, _K), lambda i: (0, 0)),
    ],
    out_specs=[
        pl.BlockSpec((_BT,), lambda i: (i,)),
        pl.BlockSpec((1, 1), lambda i: (0, 0)),
    ],
    out_shape=[
        jax.ShapeDtypeStruct((_N,), jnp.int32),
        jax.ShapeDtypeStruct((1, 1), jnp.float32),
    ],
    compiler_params=pltpu.CompilerParams(dimension_semantics=("arbitrary",)),
)


def _sc_gather_body(table_hbm, idx_hbm, out_hbm, idx_v, rows_v, sem):
    wid = lax.axis_index("s") * _NC + lax.axis_index("c")
    base = wid * _BW
    pltpu.sync_copy(idx_hbm.at[pl.ds(base, _BW)], idx_v)
    copies = [
        pltpu.async_copy(table_hbm.at[idx_v.at[pl.ds(j * _CH, _CH)]],
                         rows_v.at[j], sem)
        for j in range(_NCH)
    ]
    for j, cp in enumerate(copies):
        cp.wait()
        pltpu.sync_copy(rows_v.at[j],
                        out_hbm.at[pl.ds(base + j * _CH, _CH)])


import functools


@functools.lru_cache(maxsize=1)
def _sc_gather():
    return pl.kernel(
        _sc_gather_body,
        mesh=plsc.VectorSubcoreMesh(core_axis_name="c", subcore_axis_name="s"),
        out_type=jax.ShapeDtypeStruct((_N, 2 * _D), jnp.float32),
        scratch_types=[
            pltpu.VMEM((_BW,), jnp.int32),
            pltpu.VMEM((_NCH, _CH, 2 * _D), jnp.float32),
            pltpu.SemaphoreType.DMA,
        ],
    )


def kernel(inputs, codebook):
    cn = jnp.sum(codebook ** 2, axis=1)[None, :]
    idx, acc = _vq_tc(inputs, codebook, cn)
    cb_pad = jnp.pad(codebook, ((0, 0), (0, _D)))
    quantized = _sc_gather()(cb_pad, idx)[:, :_D]
    mean_d = acc[0, 0] / (_N * _D)
    loss = mean_d + _COMMIT * mean_d
    return quantized, loss, idx


# speedup vs baseline: 1.1728x; 1.0193x over previous
import jax
import jax.numpy as jnp
from jax import lax
from jax.experimental import pallas as pl
from jax.experimental.pallas import tpu as pltpu
from jax.experimental.pallas import tpu_sc as plsc

_K = 1024
_D = 64
_N = 16384
_COMMIT = 0.25

_BT = 4096
_NB = _N // _BT

_NC, _NS = 2, 16
_NW = _NC * _NS
_BW = _N // _NW
_CH = 128
_NCH = _BW // _CH


def _vq_tc_body(x_ref, c_ref, cn_ref, idx_ref, acc_ref):
    x = x_ref[...]
    c = c_ref[...]
    rn = jnp.sum(x * x, axis=1, keepdims=True)
    cn = cn_ref[0, :]
    mm = lax.dot_general(x, c, (((1,), (1,)), ((), ())),
                         preferred_element_type=jnp.float32)
    d = (rn + cn[None, :]) - 2.0 * mm
    lane = lax.broadcasted_iota(jnp.int32, (128, 128), 1).astype(jnp.float32)
    idx_parts = []
    gsum = None
    for g in range(_BT // 128):
        r0 = g * 128
        bestv = lax.slice(d, (r0, 0), (r0 + 128, 128))
        bestc = jnp.zeros((128, 128), jnp.float32)
        for ci in range(1, _K // 128):
            dc = lax.slice(d, (r0, ci * 128), (r0 + 128, (ci + 1) * 128))
            m = dc < bestv
            bestv = jnp.minimum(dc, bestv)
            bestc = jnp.where(m, float(ci), bestc)
        cand = bestc * 128.0 + lane
        bvt = bestv.T
        gmin_t = jnp.min(bvt, axis=0, keepdims=True)
        idxg = jnp.min(jnp.where(bvt == gmin_t, cand.T, float(_K)), axis=0)
        idx_parts.append(idxg)
        s = jnp.sum(gmin_t)
        gsum = s if gsum is None else gsum + s
    idx_f = jnp.concatenate(idx_parts)
    idx_ref[...] = idx_f.astype(jnp.int32)

    @pl.when(pl.program_id(0) == 0)
    def _init():
        acc_ref[...] = jnp.zeros_like(acc_ref)

    acc_ref[...] += gsum.reshape(1, 1)


_vq_tc = pl.pallas_call(
    _vq_tc_body,
    grid=(_NB,),
    in_specs=[
        pl.BlockSpec((_BT, _D), lambda i: (i, 0)),
        pl.BlockSpec((_K, _D), lambda i: (0, 0)),
        pl.BlockSpec((1, _K), lambda i: (0, 0)),
    ],
    out_specs=[
        pl.BlockSpec((_BT,), lambda i: (i,)),
        pl.BlockSpec((1, 1), lambda i: (0, 0)),
    ],
    out_shape=[
        jax.ShapeDtypeStruct((_N,), jnp.int32),
        jax.ShapeDtypeStruct((1, 1), jnp.float32),
    ],
    compiler_params=pltpu.CompilerParams(dimension_semantics=("arbitrary",)),
)


def _sc_gather_body(table_hbm, idx_hbm, out_hbm, idx_v, rows_v, sem):
    wid = lax.axis_index("s") * _NC + lax.axis_index("c")
    base = wid * _BW
    pltpu.sync_copy(idx_hbm.at[pl.ds(base, _BW)], idx_v)
    copies = [
        pltpu.async_copy(table_hbm.at[idx_v.at[pl.ds(j * _CH, _CH)]],
                         rows_v.at[j], sem)
        for j in range(_NCH)
    ]
    for j, cp in enumerate(copies):
        cp.wait()
        pltpu.sync_copy(rows_v.at[j],
                        out_hbm.at[pl.ds(base + j * _CH, _CH)])


import functools


@functools.lru_cache(maxsize=1)
def _sc_gather():
    return pl.kernel(
        _sc_gather_body,
        mesh=plsc.VectorSubcoreMesh(core_axis_name="c", subcore_axis_name="s"),
        out_type=jax.ShapeDtypeStruct((_N, 2 * _D), jnp.float32),
        scratch_types=[
            pltpu.VMEM((_BW,), jnp.int32),
            pltpu.VMEM((_NCH, _CH, 2 * _D), jnp.float32),
            pltpu.SemaphoreType.DMA,
        ],
    )


def kernel(inputs, codebook):
    cn = jnp.sum(codebook ** 2, axis=1)[None, :]
    idx, acc = _vq_tc(inputs, codebook, cn)
    cb_pad = jnp.pad(codebook, ((0, 0), (0, _D)))
    quantized = _sc_gather()(cb_pad, idx)[:, :_D]
    mean_d = acc[0, 0] / (_N * _D)
    loss = mean_d + _COMMIT * mean_d
    return quantized, loss, idx


# speedup vs baseline: 1.1781x; 1.0046x over previous
import jax
import jax.numpy as jnp
from jax import lax
from jax.experimental import pallas as pl
from jax.experimental.pallas import tpu as pltpu
from jax.experimental.pallas import tpu_sc as plsc

_K = 1024
_D = 64
_N = 16384
_COMMIT = 0.25

_BT = 8192
_NB = _N // _BT

_NC, _NS = 2, 16
_NW = _NC * _NS
_BW = _N // _NW
_CH = 128
_NCH = _BW // _CH


def _vq_tc_body(x_ref, c_ref, cn_ref, idx_ref, acc_ref):
    x = x_ref[...]
    c = c_ref[...]
    rn = jnp.sum(x * x, axis=1, keepdims=True)
    cn = cn_ref[0, :]
    mm = lax.dot_general(x, c, (((1,), (1,)), ((), ())),
                         preferred_element_type=jnp.float32)
    d = (rn + cn[None, :]) - 2.0 * mm
    lane = lax.broadcasted_iota(jnp.int32, (128, 128), 1).astype(jnp.float32)
    idx_parts = []
    gsum = None
    for g in range(_BT // 128):
        r0 = g * 128
        bestv = lax.slice(d, (r0, 0), (r0 + 128, 128))
        bestc = jnp.zeros((128, 128), jnp.float32)
        for ci in range(1, _K // 128):
            dc = lax.slice(d, (r0, ci * 128), (r0 + 128, (ci + 1) * 128))
            m = dc < bestv
            bestv = jnp.minimum(dc, bestv)
            bestc = jnp.where(m, float(ci), bestc)
        cand = bestc * 128.0 + lane
        bvt = bestv.T
        gmin_t = jnp.min(bvt, axis=0, keepdims=True)
        idxg = jnp.min(jnp.where(bvt == gmin_t, cand.T, float(_K)), axis=0)
        idx_parts.append(idxg)
        s = jnp.sum(gmin_t)
        gsum = s if gsum is None else gsum + s
    idx_f = jnp.concatenate(idx_parts)
    idx_ref[...] = idx_f.astype(jnp.int32)

    @pl.when(pl.program_id(0) == 0)
    def _init():
        acc_ref[...] = jnp.zeros_like(acc_ref)

    acc_ref[...] += gsum.reshape(1, 1)


_vq_tc = pl.pallas_call(
    _vq_tc_body,
    grid=(_NB,),
    in_specs=[
        pl.BlockSpec((_BT, _D), lambda i: (i, 0)),
        pl.BlockSpec((_K, _D), lambda i: (0, 0)),
        pl.BlockSpec((1, _K), lambda i: (0, 0)),
    ],
    out_specs=[
        pl.BlockSpec((_BT,), lambda i: (i,)),
        pl.BlockSpec((1, 1), lambda i: (0, 0)),
    ],
    out_shape=[
        jax.ShapeDtypeStruct((_N,), jnp.int32),
        jax.ShapeDtypeStruct((1, 1), jnp.float32),
    ],
    compiler_params=pltpu.CompilerParams(dimension_semantics=("arbitrary",)),
)


def _sc_gather_body(table_hbm, idx_hbm, out_hbm, idx_v, rows_v, sem):
    wid = lax.axis_index("s") * _NC + lax.axis_index("c")
    base = wid * _BW
    pltpu.sync_copy(idx_hbm.at[pl.ds(base, _BW)], idx_v)
    copies = [
        pltpu.async_copy(table_hbm.at[idx_v.at[pl.ds(j * _CH, _CH)]],
                         rows_v.at[j], sem)
        for j in range(_NCH)
    ]
    for j, cp in enumerate(copies):
        cp.wait()
        pltpu.sync_copy(rows_v.at[j],
                        out_hbm.at[pl.ds(base + j * _CH, _CH)])


import functools


@functools.lru_cache(maxsize=1)
def _sc_gather():
    return pl.kernel(
        _sc_gather_body,
        mesh=plsc.VectorSubcoreMesh(core_axis_name="c", subcore_axis_name="s"),
        out_type=jax.ShapeDtypeStruct((_N, 2 * _D), jnp.float32),
        scratch_types=[
            pltpu.VMEM((_BW,), jnp.int32),
            pltpu.VMEM((_NCH, _CH, 2 * _D), jnp.float32),
            pltpu.SemaphoreType.DMA,
        ],
    )


def kernel(inputs, codebook):
    cn = jnp.sum(codebook ** 2, axis=1)[None, :]
    idx, acc = _vq_tc(inputs, codebook, cn)
    cb_pad = jnp.pad(codebook, ((0, 0), (0, _D)))
    quantized = _sc_gather()(cb_pad, idx)[:, :_D]
    mean_d = acc[0, 0] / (_N * _D)
    loss = mean_d + _COMMIT * mean_d
    return quantized, loss, idx


# speedup vs baseline: 1.1827x; 1.0039x over previous
import jax
import jax.numpy as jnp
from jax import lax
from jax.experimental import pallas as pl
from jax.experimental.pallas import tpu as pltpu
from jax.experimental.pallas import tpu_sc as plsc

_K = 1024
_D = 64
_N = 16384
_COMMIT = 0.25

_BT = 8192
_NB = _N // _BT

_NC, _NS = 2, 16
_NW = _NC * _NS
_BW = _N // _NW
_CH = 128
_NCH = _BW // _CH


def _vq_tc_body(x_ref, c_ref, cn_ref, idx_ref, acc_ref):
    x = x_ref[...]
    c = c_ref[...]
    rn = jnp.sum(x * x, axis=1, keepdims=True)
    cn = cn_ref[0, :]
    mm = lax.dot_general(x, c, (((1,), (1,)), ((), ())),
                         preferred_element_type=jnp.float32)
    d = (rn + cn[None, :]) - 2.0 * mm
    lane = lax.broadcasted_iota(jnp.int32, (128, 128), 1).astype(jnp.float32)
    idx_parts = []
    gsum = None
    for g in range(_BT // 128):
        r0 = g * 128
        bestv = lax.slice(d, (r0, 0), (r0 + 128, 128))
        bestc = jnp.zeros((128, 128), jnp.float32)
        for ci in range(1, _K // 128):
            dc = lax.slice(d, (r0, ci * 128), (r0 + 128, (ci + 1) * 128))
            m = dc < bestv
            bestv = jnp.minimum(dc, bestv)
            bestc = jnp.where(m, float(ci), bestc)
        cand = bestc * 128.0 + lane
        bvt = bestv.T
        gmin_t = jnp.min(bvt, axis=0, keepdims=True)
        idxg = jnp.min(jnp.where(bvt == gmin_t, cand.T, float(_K)), axis=0)
        idx_parts.append(idxg)
        s = jnp.sum(gmin_t)
        gsum = s if gsum is None else gsum + s
    idx_f = jnp.concatenate(idx_parts)
    idx_ref[...] = idx_f.astype(jnp.int32)

    @pl.when(pl.program_id(0) == 0)
    def _init():
        acc_ref[...] = jnp.zeros_like(acc_ref)

    acc_ref[...] += gsum.reshape(1, 1)


_vq_tc = pl.pallas_call(
    _vq_tc_body,
    grid=(_NB,),
    in_specs=[
        pl.BlockSpec((_BT, _D), lambda i: (i, 0)),
        pl.BlockSpec((_K, _D), lambda i: (0, 0)),
        pl.BlockSpec((1, _K), lambda i: (0, 0)),
    ],
    out_specs=[
        pl.BlockSpec((_BT,), lambda i: (i,)),
        pl.BlockSpec((1, 1), lambda i: (0, 0)),
    ],
    out_shape=[
        jax.ShapeDtypeStruct((_N,), jnp.int32),
        jax.ShapeDtypeStruct((1, 1), jnp.float32),
    ],
    compiler_params=pltpu.CompilerParams(dimension_semantics=("arbitrary",)),
)


def _sc_gather_body(table_hbm, idx_hbm, out_hbm, idx_v, rows_v, sem, wsem):
    wid = lax.axis_index("s") * _NC + lax.axis_index("c")
    base = wid * _BW
    pltpu.sync_copy(idx_hbm.at[pl.ds(base, _BW)], idx_v)
    copies = [
        pltpu.async_copy(table_hbm.at[idx_v.at[pl.ds(j * _CH, _CH)]],
                         rows_v.at[j], sem)
        for j in range(_NCH)
    ]
    wbacks = []
    for j, cp in enumerate(copies):
        cp.wait()
        wbacks.append(
            pltpu.async_copy(rows_v.at[j],
                             out_hbm.at[pl.ds(base + j * _CH, _CH)], wsem))
    for wb in wbacks:
        wb.wait()


import functools


@functools.lru_cache(maxsize=1)
def _sc_gather():
    return pl.kernel(
        _sc_gather_body,
        mesh=plsc.VectorSubcoreMesh(core_axis_name="c", subcore_axis_name="s"),
        out_type=jax.ShapeDtypeStruct((_N, 2 * _D), jnp.float32),
        scratch_types=[
            pltpu.VMEM((_BW,), jnp.int32),
            pltpu.VMEM((_NCH, _CH, 2 * _D), jnp.float32),
            pltpu.SemaphoreType.DMA,
            pltpu.SemaphoreType.DMA,
        ],
    )


def kernel(inputs, codebook):
    cn = jnp.sum(codebook ** 2, axis=1)[None, :]
    idx, acc = _vq_tc(inputs, codebook, cn)
    cb_pad = jnp.pad(codebook, ((0, 0), (0, _D)))
    quantized = _sc_gather()(cb_pad, idx)[:, :_D]
    mean_d = acc[0, 0] / (_N * _D)
    loss = mean_d + _COMMIT * mean_d
    return quantized, loss, idx


# speedup vs baseline: 1.1971x; 1.0122x over previous
import jax
import jax.numpy as jnp
from jax import lax
from jax.experimental import pallas as pl
from jax.experimental.pallas import tpu as pltpu
from jax.experimental.pallas import tpu_sc as plsc

_K = 1024
_D = 64
_N = 16384
_COMMIT = 0.25

_BT = 8192
_NB = _N // _BT

_NC, _NS = 2, 16
_NW = _NC * _NS
_BW = _N // _NW
_CH = 128
_NCH = _BW // _CH


def _vq_tc_body(x_ref, c2_ref, cn_ref, idx_ref, acc_ref):
    x = x_ref[...]
    c2 = c2_ref[...]
    rn = jnp.sum(x * x, axis=1, keepdims=True)
    cn = cn_ref[0, :]
    mm2 = lax.dot_general(x, c2, (((1,), (1,)), ((), ())),
                          preferred_element_type=jnp.float32)
    d = (rn + cn[None, :]) - mm2
    lane = lax.broadcasted_iota(jnp.int32, (128, 128), 1).astype(jnp.float32)
    idx_parts = []
    gsum = None
    for g in range(_BT // 128):
        r0 = g * 128
        bestv = lax.slice(d, (r0, 0), (r0 + 128, 128))
        bestc = jnp.zeros((128, 128), jnp.float32)
        for ci in range(1, _K // 128):
            dc = lax.slice(d, (r0, ci * 128), (r0 + 128, (ci + 1) * 128))
            m = dc < bestv
            bestv = jnp.minimum(dc, bestv)
            bestc = jnp.where(m, float(ci), bestc)
        cand = bestc * 128.0 + lane
        bvt = bestv.T
        gmin_t = jnp.min(bvt, axis=0, keepdims=True)
        idxg = jnp.min(jnp.where(bvt == gmin_t, cand.T, float(_K)), axis=0)
        idx_parts.append(idxg)
        s = jnp.sum(gmin_t)
        gsum = s if gsum is None else gsum + s
    idx_f = jnp.concatenate(idx_parts)
    idx_ref[...] = idx_f.astype(jnp.int32)

    @pl.when(pl.program_id(0) == 0)
    def _init():
        acc_ref[...] = jnp.zeros_like(acc_ref)

    acc_ref[...] += gsum.reshape(1, 1)


_vq_tc = pl.pallas_call(
    _vq_tc_body,
    grid=(_NB,),
    in_specs=[
        pl.BlockSpec((_BT, _D), lambda i: (i, 0)),
        pl.BlockSpec((_K, _D), lambda i: (0, 0)),
        pl.BlockSpec((1, _K), lambda i: (0, 0)),
    ],
    out_specs=[
        pl.BlockSpec((_BT,), lambda i: (i,)),
        pl.BlockSpec((1, 1), lambda i: (0, 0)),
    ],
    out_shape=[
        jax.ShapeDtypeStruct((_N,), jnp.int32),
        jax.ShapeDtypeStruct((1, 1), jnp.float32),
    ],
    compiler_params=pltpu.CompilerParams(dimension_semantics=("arbitrary",)),
)


def _sc_gather_body(table_hbm, idx_hbm, out_hbm, idx_v, rows_v, sem, wsem):
    wid = lax.axis_index("s") * _NC + lax.axis_index("c")
    base = wid * _BW
    pltpu.sync_copy(idx_hbm.at[pl.ds(base, _BW)], idx_v)
    copies = [
        pltpu.async_copy(table_hbm.at[idx_v.at[pl.ds(j * _CH, _CH)]],
                         rows_v.at[j], sem)
        for j in range(_NCH)
    ]
    wbacks = []
    for j, cp in enumerate(copies):
        cp.wait()
        wbacks.append(
            pltpu.async_copy(rows_v.at[j],
                             out_hbm.at[pl.ds(base + j * _CH, _CH)], wsem))
    for wb in wbacks:
        wb.wait()


import functools


@functools.lru_cache(maxsize=1)
def _sc_gather():
    return pl.kernel(
        _sc_gather_body,
        mesh=plsc.VectorSubcoreMesh(core_axis_name="c", subcore_axis_name="s"),
        out_type=jax.ShapeDtypeStruct((_N, 2 * _D), jnp.float32),
        scratch_types=[
            pltpu.VMEM((_BW,), jnp.int32),
            pltpu.VMEM((_NCH, _CH, 2 * _D), jnp.float32),
            pltpu.SemaphoreType.DMA,
            pltpu.SemaphoreType.DMA,
        ],
    )


def kernel(inputs, codebook):
    cn = jnp.sum(codebook ** 2, axis=1)[None, :]
    idx, acc = _vq_tc(inputs, codebook + codebook, cn)
    cb_pad = jnp.pad(codebook, ((0, 0), (0, _D)))
    quantized = _sc_gather()(cb_pad, idx)[:, :_D]
    mean_d = acc[0, 0] / (_N * _D)
    loss = mean_d + _COMMIT * mean_d
    return quantized, loss, idx
